# baseline (device time: 147858 ns/iter reference)
import jax
import jax.numpy as jnp
from jax import lax
from jax.experimental import pallas as pl
from jax.experimental.pallas import tpu as pltpu

N_DEV = 16
PLANE = 4
B = 2
SQ = 512
SKV = 512
HQ_PER = 8
DH = 64
DMODEL = 768
DQ_PER = HQ_PER * DH
ROWS = B * SQ
QROWS = ROWS // PLANE
HALF = QROWS // 2

_sem_signal = getattr(pl, "semaphore_signal", None) or pltpu.semaphore_signal
_sem_wait = getattr(pl, "semaphore_wait", None) or pltpu.semaphore_wait
_run_scoped = getattr(pl, "run_scoped", None) or pltpu.run_scoped
_CompilerParams = getattr(pltpu, "CompilerParams", None) or pltpu.TPUCompilerParams

_MESH = pl.DeviceIdType.MESH


def kernel(x, Wq, K_ext, V_ext, Wo):
    my = lax.axis_index("i")
    pad = [(0, 0), (0, 0), (0, 0), (0, DH)]
    K = jnp.pad(lax.dynamic_slice_in_dim(K_ext, my * HQ_PER, HQ_PER, axis=2),
                pad)
    V = jnp.pad(lax.dynamic_slice_in_dim(V_ext, my * HQ_PER, HQ_PER, axis=2),
                pad)
    x = x.astype(jnp.bfloat16)
    Wq = Wq.astype(jnp.bfloat16)
    Wo = Wo.astype(jnp.bfloat16)

    def body(x_ref, wq_ref, k_ref, v_ref, wo_ref, out_ref,
             acc_ref, q_ref, ctx_ref,
             sendA_ref, sendB_ref, sendZ_ref,
             recvA_ref, recvB_ref, agA_ref, agB_ref,
             rz1_ref, rz2_ref, rz3_ref, rz4_ref,
             sA_sem, sB_sem, sZ_sem,
             rsA_sems, rsB_sems, z_sems, agA_sems, agB_sems):
        p = lax.axis_index("i")
        z = p // PLANE
        r = jnp.mod(p, PLANE)
        right = z * PLANE + jnp.mod(r + 1, PLANE)
        left = z * PLANE + jnp.mod(r - 1, PLANE)
        p1 = jnp.bitwise_xor(z, 1) * PLANE + r
        p2 = jnp.bitwise_xor(z, 2) * PLANE + r
        peers = (left, right, p1, p2)

        barrier = pltpu.get_barrier_semaphore()
        for nbr in peers:
            _sem_signal(barrier, inc=1, device_id=(nbr,),
                        device_id_type=_MESH)
        _sem_wait(barrier, len(peers))

        xb = x_ref[...].reshape(ROWS, DMODEL)
        q_ref[...] = jnp.dot(xb, wq_ref[...],
                             preferred_element_type=jnp.float32
                             ).astype(jnp.bfloat16)

        ri = lax.broadcasted_iota(jnp.int32, (SQ, SKV), 0) // 64
        ci = lax.broadcasted_iota(jnp.int32, (SQ, SKV), 1) // 64
        mask = ci <= ri

        for b in range(B):
            for h in range(HQ_PER):
                q = q_ref[b * SQ:(b + 1) * SQ, h * DH:(h + 1) * DH]
                k = k_ref[b, :, h, 0:DH].astype(jnp.bfloat16)
                s = lax.dot_general(
                    q, k, (((1,), (1,)), ((), ())),
                    preferred_element_type=jnp.float32) * 0.125
                w = jnp.where(mask, jnp.exp(s), 0.0)
                w = w * (1.0 / jnp.sum(w, axis=1, keepdims=True))
                v = v_ref[b, :, h, 0:DH].astype(jnp.bfloat16)
                ctx = lax.dot_general(
                    w.astype(jnp.bfloat16), v, (((1,), (0,)), ((), ())),
                    preferred_element_type=jnp.float32)
                ctx_ref[b * SQ:(b + 1) * SQ,
                        h * DH:(h + 1) * DH] = ctx.astype(jnp.bfloat16)

        acc_ref[...] = jnp.dot(ctx_ref[...], wo_ref[...],
                               preferred_element_type=jnp.float32)

        def rdma(src, dst, ssem, rsem, dev):
            op = pltpu.make_async_remote_copy(
                src_ref=src, dst_ref=dst, send_sem=ssem, recv_sem=rsem,
                device_id=(dev,), device_id_type=_MESH)
            op.start()
            return op

        for s_ in range(PLANE - 1):
            rowA = jnp.mod(r - s_, PLANE) * QROWS
            rowB = jnp.mod(r + s_, PLANE) * QROWS + HALF
            if s_ > 0:
                accA = (acc_ref[pl.ds(rowA, HALF), :]
                        + recvA_ref[s_ - 1].astype(jnp.float32))
                acc_ref[pl.ds(rowA, HALF), :] = accA
                sendA_ref[...] = accA.astype(jnp.bfloat16)
                accB = (acc_ref[pl.ds(rowB, HALF), :]
                        + recvB_ref[s_ - 1].astype(jnp.float32))
                acc_ref[pl.ds(rowB, HALF), :] = accB
                sendB_ref[...] = accB.astype(jnp.bfloat16)
            else:
                sendA_ref[...] = acc_ref[pl.ds(rowA, HALF), :
                                         ].astype(jnp.bfloat16)
                sendB_ref[...] = acc_ref[pl.ds(rowB, HALF), :
                                         ].astype(jnp.bfloat16)
            opA = rdma(sendA_ref, recvA_ref.at[s_], sA_sem,
                       rsA_sems.at[s_], right)
            opB = rdma(sendB_ref, recvB_ref.at[s_], sB_sem,
                       rsB_sems.at[s_], left)
            opA.wait()
            opB.wait()
        qa = jnp.mod(r + 1, PLANE)
        qb = jnp.mod(r - 1, PLANE)
        offA = qa * QROWS
        offB = qb * QROWS + HALF
        acc_ref[pl.ds(offA, HALF), :] = (
            acc_ref[pl.ds(offA, HALF), :]
            + recvA_ref[PLANE - 2].astype(jnp.float32))
        acc_ref[pl.ds(offB, HALF), :] = (
            acc_ref[pl.ds(offB, HALF), :]
            + recvB_ref[PLANE - 2].astype(jnp.float32))

        zb1 = jnp.mod(z, 2)
        zb2 = jnp.mod(z // 2, 2)
        keep1 = jnp.where(zb1 == 0, offA, offB)
        send1 = offA + offB - keep1
        sendZ_ref[...] = acc_ref[pl.ds(send1, HALF), :].astype(jnp.bfloat16)
        rdma(sendZ_ref, rz1_ref, sZ_sem, z_sems.at[0], p1).wait()
        acc_ref[pl.ds(keep1, HALF), :] = (
            acc_ref[pl.ds(keep1, HALF), :] + rz1_ref[...].astype(jnp.float32))
        keep2 = keep1 + jnp.where(zb2 == 0, 0, HALF // 2)
        send2 = keep1 + jnp.where(zb2 == 0, HALF // 2, 0)
        sendZ_ref[pl.ds(0, HALF // 2), :] = (
            acc_ref[pl.ds(send2, HALF // 2), :].astype(jnp.bfloat16))
        rdma(sendZ_ref.at[pl.ds(0, HALF // 2)], rz2_ref, sZ_sem,
             z_sems.at[1], p2).wait()
        acc_ref[pl.ds(keep2, HALF // 2), :] = (
            acc_ref[pl.ds(keep2, HALF // 2), :]
            + rz2_ref[...].astype(jnp.float32))
        sendZ_ref[pl.ds(0, HALF // 2), :] = (
            acc_ref[pl.ds(keep2, HALF // 2), :].astype(jnp.bfloat16))
        rdma(sendZ_ref.at[pl.ds(0, HALF // 2)], rz3_ref, sZ_sem,
             z_sems.at[2], p2).wait()
        acc_ref[pl.ds(send2, HALF // 2), :] = rz3_ref[...].astype(jnp.float32)
        sendZ_ref[...] = acc_ref[pl.ds(keep1, HALF), :].astype(jnp.bfloat16)
        rdma(sendZ_ref, rz4_ref, sZ_sem, z_sems.at[3], p1).wait()
        acc_ref[pl.ds(send1, HALF), :] = rz4_ref[...].astype(jnp.float32)

        for t in range(PLANE - 1):
            if t == 0:
                sendA_ref[...] = acc_ref[pl.ds(offA, HALF), :
                                         ].astype(jnp.bfloat16)
                sendB_ref[...] = acc_ref[pl.ds(offB, HALF), :
                                         ].astype(jnp.bfloat16)
                srcA, srcB = sendA_ref, sendB_ref
            else:
                srcA, srcB = agA_ref.at[t - 1], agB_ref.at[t - 1]
            opA = rdma(srcA, agA_ref.at[t], sA_sem, agA_sems.at[t], right)
            opB = rdma(srcB, agB_ref.at[t], sB_sem, agB_sems.at[t], left)
            opA.wait()
            opB.wait()
            rowA = jnp.mod(r - t, PLANE) * QROWS
            rowB = jnp.mod(r + t, PLANE) * QROWS + HALF
            acc_ref[pl.ds(rowA, HALF), :] = agA_ref[t].astype(jnp.float32)
            acc_ref[pl.ds(rowB, HALF), :] = agB_ref[t].astype(jnp.float32)

        out_ref[0, :, :] = acc_ref[0:SQ, :]
        out_ref[1, :, :] = acc_ref[SQ:ROWS, :]

        def _exit(exit_sem):
            for nbr in peers:
                _sem_signal(exit_sem, inc=1, device_id=(nbr,),
                            device_id_type=_MESH)
            _sem_wait(exit_sem, len(peers))
        _run_scoped(_exit, pltpu.SemaphoreType.REGULAR)

    return pl.pallas_call(
        body,
        out_shape=jax.ShapeDtypeStruct((B, SQ, DMODEL), jnp.float32),
        in_specs=[pl.BlockSpec(memory_space=pltpu.VMEM)] * 5,
        out_specs=pl.BlockSpec(memory_space=pltpu.VMEM),
        scratch_shapes=[
            pltpu.VMEM((ROWS, DMODEL), jnp.float32),
            pltpu.VMEM((ROWS, DQ_PER), jnp.bfloat16),
            pltpu.VMEM((ROWS, DQ_PER), jnp.bfloat16),
            pltpu.VMEM((HALF, DMODEL), jnp.bfloat16),
            pltpu.VMEM((HALF, DMODEL), jnp.bfloat16),
            pltpu.VMEM((HALF, DMODEL), jnp.bfloat16),
            pltpu.VMEM((PLANE - 1, HALF, DMODEL), jnp.bfloat16),
            pltpu.VMEM((PLANE - 1, HALF, DMODEL), jnp.bfloat16),
            pltpu.VMEM((PLANE - 1, HALF, DMODEL), jnp.bfloat16),
            pltpu.VMEM((PLANE - 1, HALF, DMODEL), jnp.bfloat16),
            pltpu.VMEM((HALF, DMODEL), jnp.bfloat16),
            pltpu.VMEM((HALF // 2, DMODEL), jnp.bfloat16),
            pltpu.VMEM((HALF // 2, DMODEL), jnp.bfloat16),
            pltpu.VMEM((HALF, DMODEL), jnp.bfloat16),
            pltpu.SemaphoreType.DMA,
            pltpu.SemaphoreType.DMA,
            pltpu.SemaphoreType.DMA,
            pltpu.SemaphoreType.DMA((PLANE - 1,)),
            pltpu.SemaphoreType.DMA((PLANE - 1,)),
            pltpu.SemaphoreType.DMA((4,)),
            pltpu.SemaphoreType.DMA((PLANE - 1,)),
            pltpu.SemaphoreType.DMA((PLANE - 1,)),
        ],
        compiler_params=_CompilerParams(collective_id=0),
    )(x, Wq, K, V, Wo)


# device time: 116559 ns/iter; 1.2685x vs baseline; 1.2685x over previous
import jax
import jax.numpy as jnp
from jax import lax
from jax.experimental import pallas as pl
from jax.experimental.pallas import tpu as pltpu

N_DEV = 16
PLANE = 4
B = 2
SQ = 512
SKV = 512
HQ_PER = 8
DH = 64
DMODEL = 768
DQ_PER = HQ_PER * DH
ROWS = B * SQ
QROWS = ROWS // PLANE
HALF = QROWS // 2

_sem_signal = getattr(pl, "semaphore_signal", None) or pltpu.semaphore_signal
_sem_wait = getattr(pl, "semaphore_wait", None) or pltpu.semaphore_wait
_run_scoped = getattr(pl, "run_scoped", None) or pltpu.run_scoped
_CompilerParams = getattr(pltpu, "CompilerParams", None) or pltpu.TPUCompilerParams

_MESH = pl.DeviceIdType.MESH


def kernel(x, Wq, K_ext, V_ext, Wo):
    my = lax.axis_index("i")
    K = lax.dynamic_slice_in_dim(K_ext.astype(jnp.bfloat16),
                                 my * HQ_PER, HQ_PER, axis=2)
    V = lax.dynamic_slice_in_dim(V_ext.astype(jnp.bfloat16),
                                 my * HQ_PER, HQ_PER, axis=2)
    K = K.reshape(B, SKV, DQ_PER)
    V = V.reshape(B, SKV, DQ_PER)
    x = x.astype(jnp.bfloat16)
    Wq = Wq.astype(jnp.bfloat16)
    Wo = Wo.astype(jnp.bfloat16)

    def body(x_ref, wq_ref, k_ref, v_ref, wo_ref, out_ref,
             acc_ref, q_ref, ctx_ref,
             sendA_ref, sendB_ref, sendZ_ref,
             recvA_ref, recvB_ref, agA_ref, agB_ref,
             rz1_ref, rz2_ref, rz3_ref, rz4_ref,
             sA_sem, sB_sem, sZ_sem,
             rsA_sems, rsB_sems, z_sems, agA_sems, agB_sems):
        p = lax.axis_index("i")
        z = p // PLANE
        r = jnp.mod(p, PLANE)
        right = z * PLANE + jnp.mod(r + 1, PLANE)
        left = z * PLANE + jnp.mod(r - 1, PLANE)
        p1 = jnp.bitwise_xor(z, 1) * PLANE + r
        p2 = jnp.bitwise_xor(z, 2) * PLANE + r
        peers = (left, right, p1, p2)

        barrier = pltpu.get_barrier_semaphore()
        for nbr in peers:
            _sem_signal(barrier, inc=1, device_id=(nbr,),
                        device_id_type=_MESH)
        _sem_wait(barrier, len(peers))

        xb = x_ref[...].reshape(ROWS, DMODEL)
        q_ref[...] = jnp.dot(xb, wq_ref[...],
                             preferred_element_type=jnp.float32
                             ).astype(jnp.bfloat16)

        ri = lax.broadcasted_iota(jnp.int32, (SQ, SKV), 0) // 64
        ci = lax.broadcasted_iota(jnp.int32, (SQ, SKV), 1) // 64
        mask = ci <= ri

        for b in range(B):
            for h in range(HQ_PER):
                q = q_ref[b * SQ:(b + 1) * SQ, h * DH:(h + 1) * DH]
                k = k_ref[b, :, h * DH:(h + 1) * DH]
                s = lax.dot_general(
                    q, k, (((1,), (1,)), ((), ())),
                    preferred_element_type=jnp.float32) * 0.125
                w = jnp.where(mask, jnp.exp(s), 0.0)
                w = w * (1.0 / jnp.sum(w, axis=1, keepdims=True))
                v = v_ref[b, :, h * DH:(h + 1) * DH]
                ctx = lax.dot_general(
                    w.astype(jnp.bfloat16), v, (((1,), (0,)), ((), ())),
                    preferred_element_type=jnp.float32)
                ctx_ref[b * SQ:(b + 1) * SQ,
                        h * DH:(h + 1) * DH] = ctx.astype(jnp.bfloat16)

        acc_ref[...] = jnp.dot(ctx_ref[...], wo_ref[...],
                               preferred_element_type=jnp.float32)

        def rdma(src, dst, ssem, rsem, dev):
            op = pltpu.make_async_remote_copy(
                src_ref=src, dst_ref=dst, send_sem=ssem, recv_sem=rsem,
                device_id=(dev,), device_id_type=_MESH)
            op.start()
            return op

        for s_ in range(PLANE - 1):
            rowA = jnp.mod(r - s_, PLANE) * QROWS
            rowB = jnp.mod(r + s_, PLANE) * QROWS + HALF
            if s_ > 0:
                accA = (acc_ref[pl.ds(rowA, HALF), :]
                        + recvA_ref[s_ - 1].astype(jnp.float32))
                acc_ref[pl.ds(rowA, HALF), :] = accA
                sendA_ref[...] = accA.astype(jnp.bfloat16)
                accB = (acc_ref[pl.ds(rowB, HALF), :]
                        + recvB_ref[s_ - 1].astype(jnp.float32))
                acc_ref[pl.ds(rowB, HALF), :] = accB
                sendB_ref[...] = accB.astype(jnp.bfloat16)
            else:
                sendA_ref[...] = acc_ref[pl.ds(rowA, HALF), :
                                         ].astype(jnp.bfloat16)
                sendB_ref[...] = acc_ref[pl.ds(rowB, HALF), :
                                         ].astype(jnp.bfloat16)
            opA = rdma(sendA_ref, recvA_ref.at[s_], sA_sem,
                       rsA_sems.at[s_], right)
            opB = rdma(sendB_ref, recvB_ref.at[s_], sB_sem,
                       rsB_sems.at[s_], left)
            opA.wait()
            opB.wait()
        qa = jnp.mod(r + 1, PLANE)
        qb = jnp.mod(r - 1, PLANE)
        offA = qa * QROWS
        offB = qb * QROWS + HALF
        acc_ref[pl.ds(offA, HALF), :] = (
            acc_ref[pl.ds(offA, HALF), :]
            + recvA_ref[PLANE - 2].astype(jnp.float32))
        acc_ref[pl.ds(offB, HALF), :] = (
            acc_ref[pl.ds(offB, HALF), :]
            + recvB_ref[PLANE - 2].astype(jnp.float32))

        zb1 = jnp.mod(z, 2)
        zb2 = jnp.mod(z // 2, 2)
        keep1 = jnp.where(zb1 == 0, offA, offB)
        send1 = offA + offB - keep1
        sendZ_ref[...] = acc_ref[pl.ds(send1, HALF), :].astype(jnp.bfloat16)
        rdma(sendZ_ref, rz1_ref, sZ_sem, z_sems.at[0], p1).wait()
        acc_ref[pl.ds(keep1, HALF), :] = (
            acc_ref[pl.ds(keep1, HALF), :] + rz1_ref[...].astype(jnp.float32))
        keep2 = keep1 + jnp.where(zb2 == 0, 0, HALF // 2)
        send2 = keep1 + jnp.where(zb2 == 0, HALF // 2, 0)
        sendZ_ref[pl.ds(0, HALF // 2), :] = (
            acc_ref[pl.ds(send2, HALF // 2), :].astype(jnp.bfloat16))
        rdma(sendZ_ref.at[pl.ds(0, HALF // 2)], rz2_ref, sZ_sem,
             z_sems.at[1], p2).wait()
        acc_ref[pl.ds(keep2, HALF // 2), :] = (
            acc_ref[pl.ds(keep2, HALF // 2), :]
            + rz2_ref[...].astype(jnp.float32))
        sendZ_ref[pl.ds(0, HALF // 2), :] = (
            acc_ref[pl.ds(keep2, HALF // 2), :].astype(jnp.bfloat16))
        rdma(sendZ_ref.at[pl.ds(0, HALF // 2)], rz3_ref, sZ_sem,
             z_sems.at[2], p2).wait()
        acc_ref[pl.ds(send2, HALF // 2), :] = rz3_ref[...].astype(jnp.float32)
        sendZ_ref[...] = acc_ref[pl.ds(keep1, HALF), :].astype(jnp.bfloat16)
        rdma(sendZ_ref, rz4_ref, sZ_sem, z_sems.at[3], p1).wait()
        acc_ref[pl.ds(send1, HALF), :] = rz4_ref[...].astype(jnp.float32)

        for t in range(PLANE - 1):
            if t == 0:
                sendA_ref[...] = acc_ref[pl.ds(offA, HALF), :
                                         ].astype(jnp.bfloat16)
                sendB_ref[...] = acc_ref[pl.ds(offB, HALF), :
                                         ].astype(jnp.bfloat16)
                srcA, srcB = sendA_ref, sendB_ref
            else:
                srcA, srcB = agA_ref.at[t - 1], agB_ref.at[t - 1]
            opA = rdma(srcA, agA_ref.at[t], sA_sem, agA_sems.at[t], right)
            opB = rdma(srcB, agB_ref.at[t], sB_sem, agB_sems.at[t], left)
            opA.wait()
            opB.wait()
            rowA = jnp.mod(r - t, PLANE) * QROWS
            rowB = jnp.mod(r + t, PLANE) * QROWS + HALF
            acc_ref[pl.ds(rowA, HALF), :] = agA_ref[t].astype(jnp.float32)
            acc_ref[pl.ds(rowB, HALF), :] = agB_ref[t].astype(jnp.float32)

        out_ref[0, :, :] = acc_ref[0:SQ, :]
        out_ref[1, :, :] = acc_ref[SQ:ROWS, :]

        def _exit(exit_sem):
            for nbr in peers:
                _sem_signal(exit_sem, inc=1, device_id=(nbr,),
                            device_id_type=_MESH)
            _sem_wait(exit_sem, len(peers))
        _run_scoped(_exit, pltpu.SemaphoreType.REGULAR)

    return pl.pallas_call(
        body,
        out_shape=jax.ShapeDtypeStruct((B, SQ, DMODEL), jnp.float32),
        in_specs=[pl.BlockSpec(memory_space=pltpu.VMEM)] * 5,
        out_specs=pl.BlockSpec(memory_space=pltpu.VMEM),
        scratch_shapes=[
            pltpu.VMEM((ROWS, DMODEL), jnp.float32),
            pltpu.VMEM((ROWS, DQ_PER), jnp.bfloat16),
            pltpu.VMEM((ROWS, DQ_PER), jnp.bfloat16),
            pltpu.VMEM((HALF, DMODEL), jnp.bfloat16),
            pltpu.VMEM((HALF, DMODEL), jnp.bfloat16),
            pltpu.VMEM((HALF, DMODEL), jnp.bfloat16),
            pltpu.VMEM((PLANE - 1, HALF, DMODEL), jnp.bfloat16),
            pltpu.VMEM((PLANE - 1, HALF, DMODEL), jnp.bfloat16),
            pltpu.VMEM((PLANE - 1, HALF, DMODEL), jnp.bfloat16),
            pltpu.VMEM((PLANE - 1, HALF, DMODEL), jnp.bfloat16),
            pltpu.VMEM((HALF, DMODEL), jnp.bfloat16),
            pltpu.VMEM((HALF // 2, DMODEL), jnp.bfloat16),
            pltpu.VMEM((HALF // 2, DMODEL), jnp.bfloat16),
            pltpu.VMEM((HALF, DMODEL), jnp.bfloat16),
            pltpu.SemaphoreType.DMA,
            pltpu.SemaphoreType.DMA,
            pltpu.SemaphoreType.DMA,
            pltpu.SemaphoreType.DMA((PLANE - 1,)),
            pltpu.SemaphoreType.DMA((PLANE - 1,)),
            pltpu.SemaphoreType.DMA((4,)),
            pltpu.SemaphoreType.DMA((PLANE - 1,)),
            pltpu.SemaphoreType.DMA((PLANE - 1,)),
        ],
        compiler_params=_CompilerParams(collective_id=0),
    )(x, Wq, K, V, Wo)


# device time: 116377 ns/iter; 1.2705x vs baseline; 1.0016x over previous
import jax
import jax.numpy as jnp
from jax import lax
from jax.experimental import pallas as pl
from jax.experimental.pallas import tpu as pltpu

N_DEV = 16
PLANE = 4
B = 2
SQ = 512
SKV = 512
HQ_PER = 8
DH = 64
DMODEL = 768
DQ_PER = HQ_PER * DH
ROWS = B * SQ
QROWS = ROWS // PLANE
HALF = QROWS // 2

_sem_signal = getattr(pl, "semaphore_signal", None) or pltpu.semaphore_signal
_sem_wait = getattr(pl, "semaphore_wait", None) or pltpu.semaphore_wait
_run_scoped = getattr(pl, "run_scoped", None) or pltpu.run_scoped
_CompilerParams = getattr(pltpu, "CompilerParams", None) or pltpu.TPUCompilerParams

_MESH = pl.DeviceIdType.MESH


def kernel(x, Wq, K_ext, V_ext, Wo):
    my = lax.axis_index("i")
    K = lax.dynamic_slice_in_dim(K_ext.astype(jnp.bfloat16),
                                 my * HQ_PER, HQ_PER, axis=2)
    V = lax.dynamic_slice_in_dim(V_ext.astype(jnp.bfloat16),
                                 my * HQ_PER, HQ_PER, axis=2)
    K = K.reshape(B, SKV, DQ_PER)
    V = V.reshape(B, SKV, DQ_PER)
    x = x.astype(jnp.bfloat16)
    Wq = Wq.astype(jnp.bfloat16)
    Wo = Wo.astype(jnp.bfloat16)

    def body(x_ref, wq_ref, k_ref, v_ref, wo_ref, out_ref,
             acc_ref, q_ref, ctx_ref,
             sendA_ref, sendB_ref, sendZ_ref,
             recvA_ref, recvB_ref, agA_ref, agB_ref,
             rz1_ref, rz2_ref, rz3_ref, rz4_ref,
             sA_sem, sB_sem, sZ_sem,
             rsA_sems, rsB_sems, z_sems, agA_sems, agB_sems):
        p = lax.axis_index("i")
        z = p // PLANE
        r = jnp.mod(p, PLANE)
        right = z * PLANE + jnp.mod(r + 1, PLANE)
        left = z * PLANE + jnp.mod(r - 1, PLANE)
        p1 = jnp.bitwise_xor(z, 1) * PLANE + r
        p2 = jnp.bitwise_xor(z, 2) * PLANE + r
        peers = (left, right, p1, p2)

        barrier = pltpu.get_barrier_semaphore()
        for nbr in peers:
            _sem_signal(barrier, inc=1, device_id=(nbr,),
                        device_id_type=_MESH)
        _sem_wait(barrier, len(peers))

        xb = x_ref[...].reshape(ROWS, DMODEL)
        q_ref[...] = jnp.dot(xb, wq_ref[...],
                             preferred_element_type=jnp.float32
                             ).astype(jnp.bfloat16)

        ri = lax.broadcasted_iota(jnp.int32, (SQ, SKV), 0) // 64
        ci = lax.broadcasted_iota(jnp.int32, (SQ, SKV), 1) // 64
        mask = ci <= ri

        for b in range(B):
            for h in range(HQ_PER):
                q = q_ref[b * SQ:(b + 1) * SQ, h * DH:(h + 1) * DH]
                k = k_ref[b, :, h * DH:(h + 1) * DH]
                s = lax.dot_general(
                    q, k, (((1,), (1,)), ((), ())),
                    preferred_element_type=jnp.float32) * 0.125
                w = jnp.where(mask, jnp.exp(s), 0.0)
                w = w * (1.0 / jnp.sum(w, axis=1, keepdims=True))
                v = v_ref[b, :, h * DH:(h + 1) * DH]
                ctx = lax.dot_general(
                    w.astype(jnp.bfloat16), v, (((1,), (0,)), ((), ())),
                    preferred_element_type=jnp.float32)
                ctx_ref[b * SQ:(b + 1) * SQ,
                        h * DH:(h + 1) * DH] = ctx.astype(jnp.bfloat16)

        acc_ref[...] = jnp.dot(ctx_ref[...], wo_ref[...],
                               preferred_element_type=jnp.float32)

        def rdma(src, dst, ssem, rsem, dev):
            op = pltpu.make_async_remote_copy(
                src_ref=src, dst_ref=dst, send_sem=ssem, recv_sem=rsem,
                device_id=(dev,), device_id_type=_MESH)
            op.start()
            return op

        for s_ in range(PLANE - 1):
            rowA = jnp.mod(r - s_, PLANE) * QROWS
            rowB = jnp.mod(r + s_, PLANE) * QROWS + HALF
            if s_ > 0:
                accA = (acc_ref[pl.ds(rowA, HALF), :]
                        + recvA_ref[s_ - 1].astype(jnp.float32))
                acc_ref[pl.ds(rowA, HALF), :] = accA
                sendA_ref[...] = accA.astype(jnp.bfloat16)
                accB = (acc_ref[pl.ds(rowB, HALF), :]
                        + recvB_ref[s_ - 1].astype(jnp.float32))
                acc_ref[pl.ds(rowB, HALF), :] = accB
                sendB_ref[...] = accB.astype(jnp.bfloat16)
            else:
                sendA_ref[...] = acc_ref[pl.ds(rowA, HALF), :
                                         ].astype(jnp.bfloat16)
                sendB_ref[...] = acc_ref[pl.ds(rowB, HALF), :
                                         ].astype(jnp.bfloat16)
            opA = rdma(sendA_ref, recvA_ref.at[s_], sA_sem,
                       rsA_sems.at[s_], right)
            opB = rdma(sendB_ref, recvB_ref.at[s_], sB_sem,
                       rsB_sems.at[s_], left)
            opA.wait()
            opB.wait()
        qa = jnp.mod(r + 1, PLANE)
        qb = jnp.mod(r - 1, PLANE)
        offA = qa * QROWS
        offB = qb * QROWS + HALF
        acc_ref[pl.ds(offA, HALF), :] = (
            acc_ref[pl.ds(offA, HALF), :]
            + recvA_ref[PLANE - 2].astype(jnp.float32))
        acc_ref[pl.ds(offB, HALF), :] = (
            acc_ref[pl.ds(offB, HALF), :]
            + recvB_ref[PLANE - 2].astype(jnp.float32))

        zb1 = jnp.mod(z, 2)
        zb2 = jnp.mod(z // 2, 2)
        keep1 = jnp.where(zb1 == 0, offA, offB)
        send1 = offA + offB - keep1
        sendZ_ref[...] = acc_ref[pl.ds(send1, HALF), :].astype(jnp.bfloat16)
        rdma(sendZ_ref, rz1_ref, sZ_sem, z_sems.at[0], p1).wait()
        acc_ref[pl.ds(keep1, HALF), :] = (
            acc_ref[pl.ds(keep1, HALF), :] + rz1_ref[...].astype(jnp.float32))
        keep2 = keep1 + jnp.where(zb2 == 0, 0, HALF // 2)
        send2 = keep1 + jnp.where(zb2 == 0, HALF // 2, 0)
        sendZ_ref[pl.ds(0, HALF // 2), :] = (
            acc_ref[pl.ds(send2, HALF // 2), :].astype(jnp.bfloat16))
        rdma(sendZ_ref.at[pl.ds(0, HALF // 2)], rz2_ref, sZ_sem,
             z_sems.at[1], p2).wait()
        acc_ref[pl.ds(keep2, HALF // 2), :] = (
            acc_ref[pl.ds(keep2, HALF // 2), :]
            + rz2_ref[...].astype(jnp.float32))
        sendZ_ref[pl.ds(0, HALF // 2), :] = (
            acc_ref[pl.ds(keep2, HALF // 2), :].astype(jnp.bfloat16))
        rdma(sendZ_ref.at[pl.ds(0, HALF // 2)], rz3_ref, sZ_sem,
             z_sems.at[2], p2).wait()
        acc_ref[pl.ds(send2, HALF // 2), :] = rz3_ref[...].astype(jnp.float32)
        sendZ_ref[...] = acc_ref[pl.ds(keep1, HALF), :].astype(jnp.bfloat16)
        rdma(sendZ_ref, rz4_ref, sZ_sem, z_sems.at[3], p1).wait()
        acc_ref[pl.ds(send1, HALF), :] = rz4_ref[...].astype(jnp.float32)

        for t in range(PLANE - 1):
            if t == 0:
                sendA_ref[...] = acc_ref[pl.ds(offA, HALF), :
                                         ].astype(jnp.bfloat16)
                sendB_ref[...] = acc_ref[pl.ds(offB, HALF), :
                                         ].astype(jnp.bfloat16)
                srcA, srcB = sendA_ref, sendB_ref
            else:
                srcA, srcB = agA_ref.at[t - 1], agB_ref.at[t - 1]
            opA = rdma(srcA, agA_ref.at[t], sA_sem, agA_sems.at[t], right)
            opB = rdma(srcB, agB_ref.at[t], sB_sem, agB_sems.at[t], left)
            if t > 0:
                rowA = jnp.mod(r - (t - 1), PLANE) * QROWS
                rowB = jnp.mod(r + (t - 1), PLANE) * QROWS + HALF
                acc_ref[pl.ds(rowA, HALF), :] = (
                    agA_ref[t - 1].astype(jnp.float32))
                acc_ref[pl.ds(rowB, HALF), :] = (
                    agB_ref[t - 1].astype(jnp.float32))
            opA.wait()
            opB.wait()
        t_last = PLANE - 2
        rowA = jnp.mod(r - t_last, PLANE) * QROWS
        rowB = jnp.mod(r + t_last, PLANE) * QROWS + HALF
        acc_ref[pl.ds(rowA, HALF), :] = agA_ref[t_last].astype(jnp.float32)
        acc_ref[pl.ds(rowB, HALF), :] = agB_ref[t_last].astype(jnp.float32)

        out_ref[0, :, :] = acc_ref[0:SQ, :]
        out_ref[1, :, :] = acc_ref[SQ:ROWS, :]

        def _exit(exit_sem):
            for nbr in peers:
                _sem_signal(exit_sem, inc=1, device_id=(nbr,),
                            device_id_type=_MESH)
            _sem_wait(exit_sem, len(peers))
        _run_scoped(_exit, pltpu.SemaphoreType.REGULAR)

    return pl.pallas_call(
        body,
        out_shape=jax.ShapeDtypeStruct((B, SQ, DMODEL), jnp.float32),
        in_specs=[pl.BlockSpec(memory_space=pltpu.VMEM)] * 5,
        out_specs=pl.BlockSpec(memory_space=pltpu.VMEM),
        scratch_shapes=[
            pltpu.VMEM((ROWS, DMODEL), jnp.float32),
            pltpu.VMEM((ROWS, DQ_PER), jnp.bfloat16),
            pltpu.VMEM((ROWS, DQ_PER), jnp.bfloat16),
            pltpu.VMEM((HALF, DMODEL), jnp.bfloat16),
            pltpu.VMEM((HALF, DMODEL), jnp.bfloat16),
            pltpu.VMEM((HALF, DMODEL), jnp.bfloat16),
            pltpu.VMEM((PLANE - 1, HALF, DMODEL), jnp.bfloat16),
            pltpu.VMEM((PLANE - 1, HALF, DMODEL), jnp.bfloat16),
            pltpu.VMEM((PLANE - 1, HALF, DMODEL), jnp.bfloat16),
            pltpu.VMEM((PLANE - 1, HALF, DMODEL), jnp.bfloat16),
            pltpu.VMEM((HALF, DMODEL), jnp.bfloat16),
            pltpu.VMEM((HALF // 2, DMODEL), jnp.bfloat16),
            pltpu.VMEM((HALF // 2, DMODEL), jnp.bfloat16),
            pltpu.VMEM((HALF, DMODEL), jnp.bfloat16),
            pltpu.SemaphoreType.DMA,
            pltpu.SemaphoreType.DMA,
            pltpu.SemaphoreType.DMA,
            pltpu.SemaphoreType.DMA((PLANE - 1,)),
            pltpu.SemaphoreType.DMA((PLANE - 1,)),
            pltpu.SemaphoreType.DMA((4,)),
            pltpu.SemaphoreType.DMA((PLANE - 1,)),
            pltpu.SemaphoreType.DMA((PLANE - 1,)),
        ],
        compiler_params=_CompilerParams(collective_id=0),
    )(x, Wq, K, V, Wo)
